# initial kernel scaffold (unmeasured)
import functools

import jax
import jax.numpy as jnp
from jax import lax
from jax.experimental import pallas as pl
from jax.experimental.pallas import tpu as pltpu

B, SQ, SKV_LOCAL, H, D = 8, 8, 1024, 16, 128
SCALE = D ** -0.5


def kernel(Q, K, V):
    def body(q_ref, k_ref, v_ref, out_ref,
             o_acc, o_recv, ml_acc, ml_recv, send_sems, recv_sems):
        b = pl.program_id(0)

        m_cols = []
        l_cols = []
        for h in range(H):
            qh = q_ref[b, :, h, :]
            kh = k_ref[0, :, h, :]
            vh = v_ref[0, :, h, :]
            s = lax.dot_general(
                qh, kh, (((1,), (1,)), ((), ())),
                preferred_element_type=jnp.float32,
            ) * SCALE
            m = jnp.max(s, axis=1, keepdims=True)
            p = jnp.exp(s - m)
            l = jnp.sum(p, axis=1, keepdims=True)
            o = lax.dot_general(
                p, vh, (((1,), (0,)), ((), ())),
                preferred_element_type=jnp.float32,
            )
            o_acc[b, :, h, :] = o
            m_cols.append(m)
            l_cols.append(l)
        ml_acc[0, b] = jnp.concatenate(m_cols, axis=1)
        ml_acc[1, b] = jnp.concatenate(l_cols, axis=1)

        @pl.when(b == B - 1)
        def _():
            mx = lax.axis_index("x")
            my = lax.axis_index("y")
            mz = lax.axis_index("z")
            nbr = (mx, 1 - my, mz)

            barrier_sem = pltpu.get_barrier_semaphore()
            pl.semaphore_signal(
                barrier_sem, inc=1,
                device_id=nbr, device_id_type=pl.DeviceIdType.MESH,
            )
            pl.semaphore_wait(barrier_sem, 1)

            rdma_o = pltpu.make_async_remote_copy(
                src_ref=o_acc, dst_ref=o_recv,
                send_sem=send_sems.at[0], recv_sem=recv_sems.at[0],
                device_id=nbr, device_id_type=pl.DeviceIdType.MESH,
            )
            rdma_ml = pltpu.make_async_remote_copy(
                src_ref=ml_acc, dst_ref=ml_recv,
                send_sem=send_sems.at[1], recv_sem=recv_sems.at[1],
                device_id=nbr, device_id_type=pl.DeviceIdType.MESH,
            )
            rdma_o.start()
            rdma_ml.start()
            rdma_o.wait()
            rdma_ml.wait()

            m_a = ml_acc[0]
            l_a = ml_acc[1]
            m_b = ml_recv[0]
            l_b = ml_recv[1]
            m_n = jnp.maximum(m_a, m_b)
            alpha = jnp.exp(m_a - m_n)
            beta = jnp.exp(m_b - m_n)
            l_n = alpha * l_a + beta * l_b
            out_ref[...] = (
                alpha[..., None] * o_acc[...] + beta[..., None] * o_recv[...]
            ) / l_n[..., None]

    grid = (B,)
    return pl.pallas_call(
        body,
        grid=grid,
        out_shape=jax.ShapeDtypeStruct((B, SQ, H, D), jnp.float32),
        in_specs=[
            pl.BlockSpec((B, SQ, H, D), lambda b: (0, 0, 0, 0)),
            pl.BlockSpec((1, SKV_LOCAL, H, D), lambda b: (b, 0, 0, 0)),
            pl.BlockSpec((1, SKV_LOCAL, H, D), lambda b: (b, 0, 0, 0)),
        ],
        out_specs=pl.BlockSpec((B, SQ, H, D), lambda b: (0, 0, 0, 0)),
        scratch_shapes=[
            pltpu.VMEM((B, SQ, H, D), jnp.float32),
            pltpu.VMEM((B, SQ, H, D), jnp.float32),
            pltpu.VMEM((2, B, SQ, H), jnp.float32),
            pltpu.VMEM((2, B, SQ, H), jnp.float32),
            pltpu.SemaphoreType.DMA((2,)),
            pltpu.SemaphoreType.DMA((2,)),
        ],
        compiler_params=pltpu.CompilerParams(
            dimension_semantics=("arbitrary",),
            collective_id=0,
        ),
    )(Q, K, V)


# baseline (device time: 102348 ns/iter reference)
import functools

import jax
import jax.numpy as jnp
from jax import lax
from jax.experimental import pallas as pl
from jax.experimental.pallas import tpu as pltpu

B, SQ, SKV_LOCAL, H, D = 8, 8, 1024, 16, 128
SCALE = D ** -0.5


def kernel(Q, K, V):
    def body(q_ref, k_ref, v_ref, out_ref,
             o_acc, o_recv, ml_acc, ml_recv, send_sems, recv_sems):
        b = pl.program_id(0)

        m_cols = []
        l_cols = []
        for h in range(H):
            qh = q_ref[b, :, h, :]
            kh = k_ref[0, :, h, :]
            vh = v_ref[0, :, h, :]
            s = lax.dot_general(
                qh, kh, (((1,), (1,)), ((), ())),
                preferred_element_type=jnp.float32,
            ) * SCALE
            m = jnp.max(s, axis=1, keepdims=True)
            p = jnp.exp(s - m)
            l = jnp.sum(p, axis=1, keepdims=True)
            o = lax.dot_general(
                p, vh, (((1,), (0,)), ((), ())),
                preferred_element_type=jnp.float32,
            )
            o_acc[b, :, h, :] = o
            m_cols.append(m)
            l_cols.append(l)
        ml_acc[0, b] = jnp.concatenate(m_cols, axis=1)
        ml_acc[1, b] = jnp.concatenate(l_cols, axis=1)

        @pl.when(b == B - 1)
        def _():
            mx = lax.axis_index("x")
            my = lax.axis_index("y")
            mz = lax.axis_index("z")
            nbr = (mx, 1 - my, mz)

            barrier_sem = pltpu.get_barrier_semaphore()
            pl.semaphore_signal(
                barrier_sem, inc=1,
                device_id=nbr, device_id_type=pl.DeviceIdType.MESH,
            )
            pl.semaphore_wait(barrier_sem, 1)

            rdma_o = pltpu.make_async_remote_copy(
                src_ref=o_acc, dst_ref=o_recv,
                send_sem=send_sems.at[0], recv_sem=recv_sems.at[0],
                device_id=nbr, device_id_type=pl.DeviceIdType.MESH,
            )
            rdma_ml = pltpu.make_async_remote_copy(
                src_ref=ml_acc, dst_ref=ml_recv,
                send_sem=send_sems.at[1], recv_sem=recv_sems.at[1],
                device_id=nbr, device_id_type=pl.DeviceIdType.MESH,
            )
            rdma_o.start()
            rdma_ml.start()
            rdma_o.wait()
            rdma_ml.wait()

            m_a = ml_acc[0]
            l_a = ml_acc[1]
            m_b = ml_recv[0]
            l_b = ml_recv[1]
            m_n = jnp.maximum(m_a, m_b)
            alpha = jnp.exp(m_a - m_n)
            beta = jnp.exp(m_b - m_n)
            l_n = alpha * l_a + beta * l_b
            out_ref[...] = (
                alpha[..., None] * o_acc[...] + beta[..., None] * o_recv[...]
            ) / l_n[..., None]

    grid = (B,)
    return pl.pallas_call(
        body,
        grid=grid,
        out_shape=jax.ShapeDtypeStruct((B, SQ, H, D), jnp.float32),
        in_specs=[
            pl.BlockSpec((B, SQ, H, D), lambda b: (0, 0, 0, 0)),
            pl.BlockSpec((1, SKV_LOCAL, H, D), lambda b: (b, 0, 0, 0)),
            pl.BlockSpec((1, SKV_LOCAL, H, D), lambda b: (b, 0, 0, 0)),
        ],
        out_specs=pl.BlockSpec((B, SQ, H, D), lambda b: (0, 0, 0, 0)),
        scratch_shapes=[
            pltpu.VMEM((B, SQ, H, D), jnp.float32),
            pltpu.VMEM((B, SQ, H, D), jnp.float32),
            pltpu.VMEM((2, B, SQ, H), jnp.float32),
            pltpu.VMEM((2, B, SQ, H), jnp.float32),
            pltpu.SemaphoreType.DMA((2,)),
            pltpu.SemaphoreType.DMA((2,)),
        ],
        compiler_params=pltpu.CompilerParams(
            dimension_semantics=("arbitrary",),
            collective_id=0,
            vmem_limit_bytes=100 * 1024 * 1024,
        ),
    )(Q, K, V)


# device time: 34283 ns/iter; 2.9854x vs baseline; 2.9854x over previous
import jax
import jax.numpy as jnp
from jax import lax
from jax.experimental import pallas as pl
from jax.experimental.pallas import tpu as pltpu

B, SQ, SKV_LOCAL, H, D = 8, 8, 1024, 16, 128
SCALE = D ** -0.5
NP = 8

_MESH = pl.DeviceIdType.MESH


def _ring_coords(t):
    tx = jnp.where(t < 4, 0, 1)
    tz = jnp.where(t < 4, t, 7 - t)
    return tx, tz


def kernel(Q, K, V):
    def body(q_ref, k_hbm, v_hbm, out_ref,
             k_loc, v_loc, o_mine, o_peer, ml_mine, ml_peer,
             copy_sems, y_send_sems, y_recv_sems, bc_send_sems, bc_recv_sems):
        mx = lax.axis_index("x")
        my = lax.axis_index("y")
        mz = lax.axis_index("z")
        p = mx * (7 - mz) + (1 - mx) * mz
        y_nbr = (mx, 1 - my, mz)

        barrier_sem = pltpu.get_barrier_semaphore()
        pl.semaphore_signal(barrier_sem, inc=1, device_id=y_nbr,
                            device_id_type=_MESH)
        for d in range(1, NP):
            tx, tz = _ring_coords((p + d) % NP)
            pl.semaphore_signal(barrier_sem, inc=1, device_id=(tx, my, tz),
                                device_id_type=_MESH)
        pl.semaphore_wait(barrier_sem, NP)

        ck = pltpu.make_async_copy(k_hbm.at[p], k_loc, copy_sems.at[0])
        cv = pltpu.make_async_copy(v_hbm.at[p], v_loc, copy_sems.at[1])
        ck.start()
        cv.start()
        ck.wait()
        cv.wait()

        m_cols = []
        l_cols = []
        for h in range(H):
            qh = q_ref[p, :, h, :]
            kh = k_loc[:, h, :]
            vh = v_loc[:, h, :]
            s = lax.dot_general(
                qh, kh, (((1,), (1,)), ((), ())),
                preferred_element_type=jnp.float32,
            ) * SCALE
            m = jnp.max(s, axis=1, keepdims=True)
            pr = jnp.exp(s - m)
            l = jnp.sum(pr, axis=1, keepdims=True)
            o = lax.dot_general(
                pr, vh, (((1,), (0,)), ((), ())),
                preferred_element_type=jnp.float32,
            )
            o_mine[:, h, :] = o
            m_cols.append(m)
            l_cols.append(l)
        ml_mine[0] = jnp.concatenate(m_cols, axis=1)
        ml_mine[1] = jnp.concatenate(l_cols, axis=1)

        r_o = pltpu.make_async_remote_copy(
            src_ref=o_mine, dst_ref=o_peer,
            send_sem=y_send_sems.at[0], recv_sem=y_recv_sems.at[0],
            device_id=y_nbr, device_id_type=_MESH,
        )
        r_ml = pltpu.make_async_remote_copy(
            src_ref=ml_mine, dst_ref=ml_peer,
            send_sem=y_send_sems.at[1], recv_sem=y_recv_sems.at[1],
            device_id=y_nbr, device_id_type=_MESH,
        )
        r_o.start()
        r_ml.start()
        r_o.wait()
        r_ml.wait()

        m_a = ml_mine[0]
        l_a = ml_mine[1]
        m_b = ml_peer[0]
        l_b = ml_peer[1]
        m_n = jnp.maximum(m_a, m_b)
        alpha = jnp.exp(m_a - m_n)
        beta = jnp.exp(m_b - m_n)
        l_n = alpha * l_a + beta * l_b
        out_ref[p] = (
            alpha[..., None] * o_mine[...] + beta[..., None] * o_peer[...]
        ) / l_n[..., None]

        sends = []
        for d in range(1, NP):
            tx, tz = _ring_coords((p + d) % NP)
            r = pltpu.make_async_remote_copy(
                src_ref=out_ref.at[p], dst_ref=out_ref.at[p],
                send_sem=bc_send_sems.at[d - 1], recv_sem=bc_recv_sems.at[p],
                device_id=(tx, my, tz), device_id_type=_MESH,
            )
            r.start()
            sends.append(r)
        for d in range(1, NP):
            j = (p + d) % NP
            rr = pltpu.make_async_remote_copy(
                src_ref=out_ref.at[p], dst_ref=out_ref.at[j],
                send_sem=bc_send_sems.at[0], recv_sem=bc_recv_sems.at[j],
                device_id=(mx, my, mz), device_id_type=_MESH,
            )
            rr.wait_recv()
        for r in sends:
            r.wait_send()

    return pl.pallas_call(
        body,
        out_shape=jax.ShapeDtypeStruct((B, SQ, H, D), jnp.float32),
        in_specs=[
            pl.BlockSpec(memory_space=pltpu.MemorySpace.VMEM),
            pl.BlockSpec(memory_space=pl.ANY),
            pl.BlockSpec(memory_space=pl.ANY),
        ],
        out_specs=pl.BlockSpec(memory_space=pltpu.MemorySpace.VMEM),
        scratch_shapes=[
            pltpu.VMEM((SKV_LOCAL, H, D), jnp.float32),
            pltpu.VMEM((SKV_LOCAL, H, D), jnp.float32),
            pltpu.VMEM((SQ, H, D), jnp.float32),
            pltpu.VMEM((SQ, H, D), jnp.float32),
            pltpu.VMEM((2, SQ, H), jnp.float32),
            pltpu.VMEM((2, SQ, H), jnp.float32),
            pltpu.SemaphoreType.DMA((2,)),
            pltpu.SemaphoreType.DMA((2,)),
            pltpu.SemaphoreType.DMA((2,)),
            pltpu.SemaphoreType.DMA((NP - 1,)),
            pltpu.SemaphoreType.DMA((NP,)),
        ],
        compiler_params=pltpu.CompilerParams(
            collective_id=0,
            vmem_limit_bytes=100 * 1024 * 1024,
        ),
    )(Q, K, V)
